# baseline (speedup 1.0000x reference)
import functools

import jax
import jax.numpy as jnp
from jax import lax
from jax.experimental import pallas as pl
from jax.experimental.pallas import tpu as pltpu
from jax.experimental.pallas import tpu_sc as plsc

VOCAB = 1000000
DIM = 64
SCALE = 8.0
NC = 2
NS = 16
NW = NC * NS
LANES = 16

N_FULL_BLK = VOCAB // 128
TAIL = VOCAB - N_FULL_BLK * 128
BLK_PER_W = N_FULL_BLK // NW + 1


def _mesh():
    return plsc.VectorSubcoreMesh(
        core_axis_name="c", subcore_axis_name="s", num_cores=NC, num_subcores=NS
    )


def _iota16():
    return lax.iota(jnp.int32, LANES)


@functools.lru_cache(maxsize=None)
def _build_relayout():
    @functools.partial(
        pl.kernel,
        out_type=jax.ShapeDtypeStruct((VOCAB, 128), jnp.float32),
        mesh=_mesh(),
        scratch_types=[
            pltpu.VMEM((DIM, 128), jnp.float32),
            pltpu.VMEM((128, 128), jnp.float32),
        ],
        compiler_params=pltpu.CompilerParams(needs_layout_passes=False),
    )
    def relayout(tabT, tailp, tab2, gbuf, tbuf):
        wid = lax.axis_index("s") * NC + lax.axis_index("c")

        def do_block(g, _):
            @pl.when(g < N_FULL_BLK)
            def _():
                v0 = g * 128
                pltpu.sync_copy(tabT.at[:, pl.ds(v0, 128)], gbuf)

                def tr_row(vp, c):
                    col = jnp.full((LANES,), 0, jnp.int32) + vp
                    for q in range(DIM // LANES):
                        rows = _iota16() + (q * LANES)
                        vals = plsc.load_gather(gbuf, [rows, col])
                        tbuf[vp, pl.ds(q * LANES, LANES)] = vals
                    return c

                lax.fori_loop(0, 128, tr_row, 0)
                pltpu.sync_copy(tbuf, tab2.at[pl.ds(v0, 128), :])

            return _

        lax.fori_loop(0, BLK_PER_W, lambda t, c: (do_block(t * NW + wid, c), 0)[1], 0)

        @pl.when(wid == NW - 1)
        def _tail():
            v0 = N_FULL_BLK * 128
            pltpu.sync_copy(tailp, gbuf)

            def tr_row(vp, c):
                col = jnp.full((LANES,), 0, jnp.int32) + vp
                for q in range(DIM // LANES):
                    rows = _iota16() + (q * LANES)
                    vals = plsc.load_gather(gbuf, [rows, col])
                    tbuf[vp, pl.ds(q * LANES, LANES)] = vals
                return c

            lax.fori_loop(0, TAIL, tr_row, 0)
            pltpu.sync_copy(tbuf.at[pl.ds(0, TAIL), :], tab2.at[pl.ds(v0, TAIL), :])

    return relayout


N_SBLK = 200 // 8
N_BBLK = 4096 // 128
UNITS_PER_W = N_SBLK * N_BBLK // NW


@functools.lru_cache(maxsize=None)
def _build_gather():
    @functools.partial(
        pl.kernel,
        out_type=jax.ShapeDtypeStruct((200, DIM, 4096), jnp.float32),
        mesh=_mesh(),
        scratch_types=[
            pltpu.VMEM((8, 128), jnp.int32),
            pltpu.VMEM((128, 128), jnp.float32),
            pltpu.VMEM((DIM, 128), jnp.float32),
            pltpu.SemaphoreType.DMA,
        ],
        compiler_params=pltpu.CompilerParams(needs_layout_passes=False),
    )
    def gather(valsT, tab2, out3, idx_v, gbuf2, trans2, sem):
        wid = lax.axis_index("s") * NC + lax.axis_index("c")

        def do_unit(u, _):
            g = wid * UNITS_PER_W + u
            sb = g // N_BBLK
            bb = g - sb * N_BBLK
            s0 = sb * 8
            b0 = bb * 128
            pltpu.sync_copy(valsT.at[pl.ds(s0, 8), pl.ds(b0, 128)], idx_v)

            def do_slab(j, c):
                pltpu.async_copy(tab2.at[idx_v.at[j]], gbuf2, sem).wait()

                def tr(g8, c2):
                    rows = _iota16() + g8 * LANES
                    for d in range(DIM):
                        col = jnp.full((LANES,), d, jnp.int32)
                        vals = plsc.load_gather(gbuf2, [rows, col])
                        trans2[d, pl.ds(g8 * LANES, LANES)] = vals * SCALE
                    return c2

                lax.fori_loop(0, 128 // LANES, tr, 0)
                pltpu.sync_copy(trans2, out3.at[s0 + j, :, pl.ds(b0, 128)])
                return c

            lax.fori_loop(0, 8, do_slab, 0)
            return _

        lax.fori_loop(0, UNITS_PER_W, do_unit, 0)

    return gather


def kernel(values, table):
    tabT = table.T
    valsT = values.T
    tailp = jnp.pad(table[N_FULL_BLK * 128:].T, ((0, 0), (0, 128 - TAIL)))
    tab2 = _build_relayout()(tabT, tailp)
    out3 = _build_gather()(valsT, tab2)
    return out3.transpose(2, 0, 1)


# speedup vs baseline: 2.3870x; 2.3870x over previous
import functools

import jax
import jax.numpy as jnp
from jax import lax
from jax.experimental import pallas as pl
from jax.experimental.pallas import tpu as pltpu
from jax.experimental.pallas import tpu_sc as plsc

VOCAB = 1000000
DIM = 64
SCALE = 8.0
NC = 2
NS = 16
NW = NC * NS
LANES = 16

N_FULL_BLK = VOCAB // 128
TAIL = VOCAB - N_FULL_BLK * 128
PAIRS = (N_FULL_BLK // NW + 2) // 2


def _mesh():
    return plsc.VectorSubcoreMesh(
        core_axis_name="c", subcore_axis_name="s", num_cores=NC, num_subcores=NS
    )


def _iota16():
    return lax.iota(jnp.int32, LANES)


def _transpose_block(src, dst, n_rows):
    rows = [_iota16() + (q * LANES) for q in range(DIM // LANES)]

    @plsc.parallel_loop(0, n_rows, 1, unroll=8)
    def _(vp):
        col = jnp.full((LANES,), 0, jnp.int32) + vp
        for q in range(DIM // LANES):
            vals = plsc.load_gather(src, [rows[q], col])
            dst[vp, pl.ds(q * LANES, LANES)] = vals


@functools.lru_cache(maxsize=None)
def _build_relayout():
    @functools.partial(
        pl.kernel,
        out_type=jax.ShapeDtypeStruct((VOCAB, 128), jnp.float32),
        mesh=_mesh(),
        scratch_types=[
            pltpu.VMEM((DIM, 128), jnp.float32),
            pltpu.VMEM((DIM, 128), jnp.float32),
            pltpu.VMEM((128, 128), jnp.float32),
            pltpu.VMEM((128, 128), jnp.float32),
            pltpu.SemaphoreType.DMA,
            pltpu.SemaphoreType.DMA,
            pltpu.SemaphoreType.DMA,
            pltpu.SemaphoreType.DMA,
        ],
        compiler_params=pltpu.CompilerParams(needs_layout_passes=False),
    )
    def relayout(tabT, tailp, tab2, gb0, gb1, tb0, tb1, si0, si1, so0, so1):
        wid = lax.axis_index("s") * NC + lax.axis_index("c")
        gbufs, tbufs = (gb0, gb1), (tb0, tb1)
        sins, souts = (si0, si1), (so0, so1)

        def blk(t):
            return t * NW + wid

        def start_in(t, p):
            @pl.when(blk(t) < N_FULL_BLK)
            def _():
                pltpu.async_copy(
                    tabT.at[:, pl.ds(blk(t) * 128, 128)], gbufs[p], sins[p]
                )

        start_in(0, 0)

        def pair(t2, c):
            for p in (0, 1):
                t = t2 * 2 + p
                g = blk(t)
                start_in(t + 1, 1 - p)

                @pl.when(g < N_FULL_BLK)
                def _():
                    pltpu.make_async_copy(
                        tabT.at[:, pl.ds(g * 128, 128)], gbufs[p], sins[p]
                    ).wait()

                    @pl.when(t >= 2)
                    def _w():
                        pltpu.make_async_copy(
                            tbufs[p], tab2.at[pl.ds(0, 128), :], souts[p]
                        ).wait()

                    _transpose_block(gbufs[p], tbufs[p], 128)
                    pltpu.async_copy(
                        tbufs[p], tab2.at[pl.ds(g * 128, 128), :], souts[p]
                    )

            return c

        lax.fori_loop(0, PAIRS, pair, 0)
        for p in (0, 1):
            pltpu.make_async_copy(
                tbufs[p], tab2.at[pl.ds(0, 128), :], souts[p]
            ).wait()

        @pl.when(wid == NW - 1)
        def _tail():
            v0 = N_FULL_BLK * 128
            pltpu.sync_copy(tailp, gb0)
            _transpose_block(gb0, tb0, TAIL)
            pltpu.sync_copy(tb0.at[pl.ds(0, TAIL), :], tab2.at[pl.ds(v0, TAIL), :])

    return relayout


N_SBLK = 200 // 8
N_BBLK = 4096 // 128
UNITS_PER_W = N_SBLK * N_BBLK // NW


@functools.lru_cache(maxsize=None)
def _build_gather():
    @functools.partial(
        pl.kernel,
        out_type=jax.ShapeDtypeStruct((200, DIM, 4096), jnp.float32),
        mesh=_mesh(),
        scratch_types=[
            pltpu.VMEM((8, 128), jnp.int32),
            pltpu.VMEM((128, 128), jnp.float32),
            pltpu.VMEM((128, 128), jnp.float32),
            pltpu.VMEM((DIM, 128), jnp.float32),
            pltpu.VMEM((DIM, 128), jnp.float32),
            pltpu.SemaphoreType.DMA,
            pltpu.SemaphoreType.DMA,
            pltpu.SemaphoreType.DMA,
            pltpu.SemaphoreType.DMA,
        ],
        compiler_params=pltpu.CompilerParams(needs_layout_passes=False),
    )
    def gather(valsT, tab2, out3, idx_v, gb0, gb1, tr0, tr1, g0, g1, o0, o1):
        wid = lax.axis_index("s") * NC + lax.axis_index("c")
        gbufs, trbufs = (gb0, gb1), (tr0, tr1)
        gsems, osems = (g0, g1), (o0, o1)
        rows = [_iota16() + (g8 * LANES) for g8 in range(128 // LANES)]

        def do_unit(u, c):
            g = wid * UNITS_PER_W + u
            sb = g // N_BBLK
            bb = g - sb * N_BBLK
            s0 = sb * 8
            b0 = bb * 128
            pltpu.sync_copy(valsT.at[pl.ds(s0, 8), pl.ds(b0, 128)], idx_v)

            pltpu.async_copy(tab2.at[idx_v.at[0]], gbufs[0], gsems[0])
            for j in range(8):
                p = j % 2
                if j + 1 < 8:
                    pltpu.async_copy(
                        tab2.at[idx_v.at[j + 1]], gbufs[1 - p], gsems[1 - p]
                    )
                pltpu.make_async_copy(
                    tab2.at[idx_v.at[j]], gbufs[p], gsems[p]
                ).wait()
                if j >= 2:
                    pltpu.make_async_copy(
                        trbufs[p], out3.at[s0, :, pl.ds(b0, 128)], osems[p]
                    ).wait()

                @plsc.parallel_loop(0, DIM, 1, unroll=8)
                def _(d):
                    col = jnp.full((LANES,), 0, jnp.int32) + d
                    for g8 in range(128 // LANES):
                        vals = plsc.load_gather(gbufs[p], [rows[g8], col])
                        trbufs[p][d, pl.ds(g8 * LANES, LANES)] = vals * SCALE

                pltpu.async_copy(
                    trbufs[p], out3.at[s0 + j, :, pl.ds(b0, 128)], osems[p]
                )

            for p in (0, 1):
                pltpu.make_async_copy(
                    trbufs[p], out3.at[s0, :, pl.ds(b0, 128)], osems[p]
                ).wait()
            return c

        lax.fori_loop(0, UNITS_PER_W, do_unit, 0)

    return gather


def kernel(values, table):
    tabT = table.T
    valsT = values.T
    tailp = jnp.pad(table[N_FULL_BLK * 128:].T, ((0, 0), (0, 128 - TAIL)))
    tab2 = _build_relayout()(tabT, tailp)
    out3 = _build_gather()(valsT, tab2)
    return out3.transpose(2, 0, 1)
